# baseline (speedup 1.0000x reference)
import functools

import jax
import jax.numpy as jnp
from jax import lax
from jax.experimental import pallas as pl
from jax.experimental.pallas import tpu as pltpu
from jax.experimental.pallas import tpu_sc as plsc

B = 4096
L = 50
EMBED = 128
HID = 64
LANES = 16
NC = 2
NS = 16
NW = NC * NS
BPW = B // NW
PAIRS = BPW // 2
PL2 = 2 * L
NCH = HID // LANES


def _fold_body(t_ref, w_ref, o_ref):
    o_ref[...] = jnp.dot(t_ref[...], w_ref[...],
                         preferred_element_type=jnp.float32)


def _fold(table, w1_part, blk):
    v = table.shape[0]
    return pl.pallas_call(
        _fold_body,
        grid=(v // blk,),
        in_specs=[
            pl.BlockSpec((blk, EMBED), lambda i: (i, 0)),
            pl.BlockSpec((EMBED, HID), lambda i: (0, 0)),
        ],
        out_specs=pl.BlockSpec((blk, HID), lambda i: (i, 0)),
        out_shape=jax.ShapeDtypeStruct((v, HID), jnp.float32),
    )(table, w1_part)


def _sc_gather_pool(diag_codes, proc_codes, med_codes, f_diag, f_proc, f_med):
    mesh = plsc.VectorSubcoreMesh(core_axis_name="c", subcore_axis_name="s")

    @functools.partial(
        pl.kernel,
        mesh=mesh,
        compiler_params=pltpu.CompilerParams(use_tc_tiling_on_sc=False),
        out_type=jax.ShapeDtypeStruct((B, HID), jnp.float32),
        scratch_types=[
            pltpu.VMEM((PAIRS, PL2), jnp.int32),
            pltpu.VMEM((PAIRS, PL2), jnp.int32),
            pltpu.VMEM((PAIRS, PL2), jnp.int32),
            pltpu.VMEM((2, 3, PL2, HID), jnp.float32),
            pltpu.VMEM((BPW, HID), jnp.float32),
            pltpu.SemaphoreType.DMA,
            pltpu.SemaphoreType.DMA,
        ],
    )
    def k(diag_hbm, proc_hbm, med_hbm, fd_hbm, fp_hbm, fm_hbm, out_hbm,
          idx_d, idx_p, idx_m, rows, out_v, sem0, sem1):
        wid = lax.axis_index("s") * NC + lax.axis_index("c")
        pbase = wid * PAIRS
        sems = (sem0, sem1)
        tables = (fd_hbm, fp_hbm, fm_hbm)
        idxs = (idx_d, idx_p, idx_m)

        pltpu.sync_copy(diag_hbm.at[pl.ds(pbase, PAIRS)], idx_d)
        pltpu.sync_copy(proc_hbm.at[pl.ds(pbase, PAIRS)], idx_p)
        pltpu.sync_copy(med_hbm.at[pl.ds(pbase, PAIRS)], idx_m)

        def fire(p, slot):
            sem = sems[slot]
            for t in range(3):
                pltpu.make_async_copy(
                    tables[t].at[idxs[t].at[p]], rows.at[slot, t], sem).start()

        def drain(slot):
            sem = sems[slot]
            for t in range(3):
                pltpu.make_async_copy(
                    tables[t].at[idxs[t].at[0]], rows.at[slot, t], sem).wait()

        fire(0, 0)
        fire(1, 1)

        def pair_body(p2, _):
            for slot in range(2):
                p = p2 * 2 + slot
                drain(slot)

                def row_body(r, accs, slot=slot):
                    new = list(accs)
                    for t in range(3):
                        for c in range(NCH):
                            sl = pl.ds(c * LANES, LANES)
                            new[c] = new[c] + rows[slot, t, r, sl]
                            new[NCH + c] = new[NCH + c] + rows[slot, t, L + r, sl]
                    return tuple(new)

                accs = lax.fori_loop(
                    0, L, row_body,
                    tuple(jnp.zeros((LANES,), jnp.float32)
                          for _ in range(2 * NCH)))
                for c in range(NCH):
                    out_v[2 * p, pl.ds(c * LANES, LANES)] = accs[c]
                    out_v[2 * p + 1, pl.ds(c * LANES, LANES)] = accs[NCH + c]

                @pl.when(p + 2 < PAIRS)
                def _(p=p, slot=slot):
                    fire(p + 2, slot)
            return 0

        lax.fori_loop(0, PAIRS // 2, pair_body, 0)

        pltpu.sync_copy(out_v, out_hbm.at[pl.ds(wid * BPW, BPW)])

    return k(diag_codes, proc_codes, med_codes, f_diag, f_proc, f_med)


def _tail_body(h_ref, b1_ref, w2_ref, b2_ref, o_ref):
    h = jnp.maximum(h_ref[...] + b1_ref[...], 0.0)
    z = jnp.dot(h, w2_ref[...], preferred_element_type=jnp.float32)
    o_ref[...] = jax.nn.sigmoid(z + b2_ref[...])


def _tc_tail(hsum, b1, W2, b2):
    blk = 512
    nout = W2.shape[1]
    return pl.pallas_call(
        _tail_body,
        grid=(B // blk,),
        in_specs=[
            pl.BlockSpec((blk, HID), lambda i: (i, 0)),
            pl.BlockSpec((1, HID), lambda i: (0, 0)),
            pl.BlockSpec((HID, nout), lambda i: (0, 0)),
            pl.BlockSpec((1, nout), lambda i: (0, 0)),
        ],
        out_specs=pl.BlockSpec((blk, nout), lambda i: (i, 0)),
        out_shape=jax.ShapeDtypeStruct((B, nout), jnp.float32),
    )(hsum, b1.reshape(1, -1), W2, b2.reshape(1, -1))


def kernel(diag_codes, proc_codes, prev_med_codes, W_diag, W_proc, W_med,
           W1, b1, W2, b2):
    f_diag = _fold(W_diag, W1[:EMBED], 5000)
    f_proc = _fold(W_proc, W1[EMBED:2 * EMBED], 5000)
    f_med = _fold(W_med, W1[2 * EMBED:], 1000)
    hsum = _sc_gather_pool(
        diag_codes.reshape(B // 2, PL2),
        proc_codes.reshape(B // 2, PL2),
        prev_med_codes.reshape(B // 2, PL2),
        f_diag, f_proc, f_med)
    return _tc_tail(hsum, b1, W2, b2)
